# baseline (speedup 1.0000x reference)
import jax
import jax.numpy as jnp
from jax import lax
from jax.experimental import pallas as pl
from jax.experimental.pallas import tpu as pltpu

EMBED_DIM = 2048
NUM_EXPERTS = 16
TOP_K = 2

BLOCK_T = 2048
NSPLIT = 8
SUB_T = BLOCK_T // NSPLIT


def _router_block(x_hbm, w_ref, logits_ref, aux_ref, x_buf, sems):
    i = pl.program_id(0)
    nsteps = pl.num_programs(0)

    def copy(step, slot, s):
        return pltpu.make_async_copy(
            x_hbm.at[pl.ds(step * BLOCK_T + s * SUB_T, SUB_T), :],
            x_buf.at[slot, pl.ds(s * SUB_T, SUB_T), :],
            sems.at[slot, s],
        )

    slot = lax.rem(i, 2)
    nxt = lax.rem(i + 1, 2)

    @pl.when(i == 0)
    def _first():
        for s in range(NSPLIT):
            copy(0, 0, s).start()

    @pl.when(i + 1 < nsteps)
    def _prefetch():
        for s in range(NSPLIT):
            copy(i + 1, nxt, s).start()

    for s in range(NSPLIT):
        copy(i, slot, s).wait()

    logits = jax.lax.dot_general(
        x_buf[slot], w_ref[...],
        dimension_numbers=(((1,), (1,)), ((), ())),
        preferred_element_type=jnp.float32,
    )
    logits_ref[...] = logits

    iota = lax.broadcasted_iota(jnp.int32, logits.shape, 1)
    m1 = jnp.max(logits, axis=-1, keepdims=True)
    i1 = jnp.min(jnp.where(logits == m1, iota, NUM_EXPERTS), axis=-1,
                 keepdims=True)
    masked = jnp.where(iota == i1, -jnp.inf, logits)
    m2 = jnp.max(masked, axis=-1, keepdims=True)
    i2 = jnp.min(jnp.where(masked == m2, iota, NUM_EXPERTS), axis=-1,
                 keepdims=True)
    w1 = jax.nn.sigmoid(m1 - m2)
    i1f = lax.bitcast_convert_type(i1, jnp.float32)
    i2f = lax.bitcast_convert_type(i2, jnp.float32)
    aux_ref[...] = jnp.concatenate(
        [i1f, i2f, w1, 1.0 - w1, jnp.zeros_like(logits[:, :12])], axis=-1)


def kernel(x, W):
    n_tokens = x.shape[0]
    grid = (n_tokens // BLOCK_T,)
    logits, aux = pl.pallas_call(
        _router_block,
        grid=grid,
        in_specs=[
            pl.BlockSpec(memory_space=pl.ANY),
            pl.BlockSpec((NUM_EXPERTS, EMBED_DIM), lambda i: (0, 0)),
        ],
        out_specs=(
            pl.BlockSpec((BLOCK_T, NUM_EXPERTS), lambda i: (i, 0)),
            pl.BlockSpec((BLOCK_T, NUM_EXPERTS), lambda i: (i, 0)),
        ),
        out_shape=(
            jax.ShapeDtypeStruct((n_tokens, NUM_EXPERTS), jnp.float32),
            jax.ShapeDtypeStruct((n_tokens, NUM_EXPERTS), jnp.float32),
        ),
        scratch_shapes=[
            pltpu.VMEM((2, BLOCK_T, EMBED_DIM), jnp.float32),
            pltpu.SemaphoreType.DMA((2, NSPLIT)),
        ],
    )(x, W)
    idx = lax.bitcast_convert_type(aux[:, :TOP_K], jnp.int32)
    wgt = aux[:, TOP_K:2 * TOP_K]
    return (idx, wgt, logits)
